# baseline (speedup 1.0000x reference)
import functools

import jax
import jax.numpy as jnp
from jax import lax
from jax.experimental import pallas as pl
from jax.experimental.pallas import tpu as pltpu
from jax.experimental.pallas import tpu_sc as plsc

_SEQ = 200
_D = 64
_LANES = 16
_CHUNK = 128


@functools.lru_cache(maxsize=None)
def _build(n_rows: int):
    info = plsc.get_sparse_core_info()
    nw = info.num_cores * info.num_subcores
    assert n_rows % (nw * _CHUNK) == 0
    per_w = n_rows // nw
    n_chunks = per_w // _CHUNK
    assert per_w % _SEQ == 0

    mesh = plsc.VectorSubcoreMesh(core_axis_name="c", subcore_axis_name="s")

    @functools.partial(
        pl.kernel,
        mesh=mesh,
        out_type=jax.ShapeDtypeStruct((n_rows, _D), jnp.float32),
        scratch_types=[
            pltpu.VMEM((_CHUNK,), jnp.int32),
            pltpu.VMEM((_CHUNK, _D), jnp.float32),
            pltpu.VMEM((2 * _SEQ, _D), jnp.float32),
            pltpu.SemaphoreType.DMA,
        ],
        compiler_params=pltpu.CompilerParams(use_tc_tiling_on_sc=False),
    )
    def emb(x_hbm, table_hbm, pos_hbm, out_hbm, idx_v, rows_v, pos2_v, sem):
        wid = lax.axis_index("s") * info.num_cores + lax.axis_index("c")
        base = wid * per_w
        pltpu.sync_copy(pos_hbm, pos2_v.at[pl.ds(0, _SEQ)])
        pltpu.sync_copy(pos_hbm, pos2_v.at[pl.ds(_SEQ, _SEQ)])

        def chunk_body(g, carry):
            cbase = base + g * _CHUNK
            pltpu.sync_copy(x_hbm.at[pl.ds(cbase, _CHUNK)], idx_v)
            pltpu.async_copy(table_hbm.at[idx_v], rows_v, sem).wait()
            p0 = lax.rem(g * _CHUNK, _SEQ)

            def add_row(i, c):
                for j in range(_D // _LANES):
                    plsc.addupdate(
                        rows_v.at[i, pl.ds(j * _LANES, _LANES)],
                        pos2_v[p0 + i, pl.ds(j * _LANES, _LANES)],
                    )
                return c

            lax.fori_loop(0, _CHUNK, add_row, 0, unroll=4)
            pltpu.sync_copy(rows_v, out_hbm.at[pl.ds(cbase, _CHUNK)])
            return carry

        lax.fori_loop(0, n_chunks, chunk_body, 0)

    return emb


def kernel(x, token_table, pos_table):
    b, l = x.shape
    out = _build(b * l)(x.reshape(-1), token_table, pos_table)
    return out.reshape(b, l, _D)


# speedup vs baseline: 1.3129x; 1.3129x over previous
import functools

import jax
import jax.numpy as jnp
from jax import lax
from jax.experimental import pallas as pl
from jax.experimental.pallas import tpu as pltpu
from jax.experimental.pallas import tpu_sc as plsc

_SEQ = 200
_D = 64
_LANES = 16
_CHUNK = 400


@functools.lru_cache(maxsize=None)
def _build(n_rows: int):
    info = plsc.get_sparse_core_info()
    nw = info.num_cores * info.num_subcores
    assert n_rows % (nw * _CHUNK) == 0
    per_w = n_rows // nw
    n_chunks = per_w // _CHUNK
    assert per_w % _SEQ == 0

    mesh = plsc.VectorSubcoreMesh(core_axis_name="c", subcore_axis_name="s")

    @functools.partial(
        pl.kernel,
        mesh=mesh,
        out_type=jax.ShapeDtypeStruct((n_rows, _D), jnp.float32),
        scratch_types=[
            pltpu.VMEM((_CHUNK,), jnp.int32),
            pltpu.VMEM((_CHUNK, _D), jnp.float32),
            pltpu.VMEM((2 * _SEQ, _D), jnp.float32),
            pltpu.SemaphoreType.DMA,
        ],
        compiler_params=pltpu.CompilerParams(use_tc_tiling_on_sc=False),
    )
    def emb(x_hbm, table_hbm, pos_hbm, out_hbm, idx_v, rows_v, pos2_v, sem):
        wid = lax.axis_index("s") * info.num_cores + lax.axis_index("c")
        base = wid * per_w
        pltpu.sync_copy(pos_hbm, pos2_v.at[pl.ds(0, _SEQ)])
        pltpu.sync_copy(pos_hbm, pos2_v.at[pl.ds(_SEQ, _SEQ)])

        def chunk_body(g, carry):
            cbase = base + g * _CHUNK
            pltpu.sync_copy(x_hbm.at[pl.ds(cbase, _CHUNK)], idx_v)
            pltpu.async_copy(table_hbm.at[idx_v], rows_v, sem).wait()

            def add_row(i, c):
                for j in range(_D // _LANES):
                    plsc.addupdate(
                        rows_v.at[i, pl.ds(j * _LANES, _LANES)],
                        pos2_v[i, pl.ds(j * _LANES, _LANES)],
                    )
                return c

            lax.fori_loop(0, _CHUNK, add_row, 0, unroll=4)
            pltpu.sync_copy(rows_v, out_hbm.at[pl.ds(cbase, _CHUNK)])
            return carry

        lax.fori_loop(0, n_chunks, chunk_body, 0)

    return emb


def kernel(x, token_table, pos_table):
    b, l = x.shape
    out = _build(b * l)(x.reshape(-1), token_table, pos_table)
    return out.reshape(b, l, _D)


# speedup vs baseline: 1.4653x; 1.1161x over previous
import functools

import jax
import jax.numpy as jnp
from jax import lax
from jax.experimental import pallas as pl
from jax.experimental.pallas import tpu as pltpu
from jax.experimental.pallas import tpu_sc as plsc

_SEQ = 200
_D = 64
_LANES = 16
_CHUNK = 2 * _SEQ


@functools.lru_cache(maxsize=None)
def _build(n_rows: int):
    info = plsc.get_sparse_core_info()
    nw = info.num_cores * info.num_subcores
    assert n_rows % (nw * _CHUNK) == 0
    per_w = n_rows // nw
    n_chunks = per_w // _CHUNK
    assert n_chunks % 2 == 0

    mesh = plsc.VectorSubcoreMesh(core_axis_name="c", subcore_axis_name="s")

    @functools.partial(
        pl.kernel,
        mesh=mesh,
        out_type=jax.ShapeDtypeStruct((n_rows, _D), jnp.float32),
        scratch_types=[
            pltpu.VMEM((2, _CHUNK), jnp.int32),
            pltpu.VMEM((_CHUNK, _D), jnp.float32),
            pltpu.VMEM((_CHUNK, _D), jnp.float32),
            pltpu.VMEM((_CHUNK, _D), jnp.float32),
            pltpu.SemaphoreType.DMA,
            pltpu.SemaphoreType.DMA,
            pltpu.SemaphoreType.DMA,
            pltpu.SemaphoreType.DMA,
        ],
        compiler_params=pltpu.CompilerParams(use_tc_tiling_on_sc=False),
    )
    def emb(x_hbm, table_hbm, pos_hbm, out_hbm,
            idx_v, rows0_v, rows1_v, pos2_v,
            gsem0, gsem1, isem0, isem1):
        wid = lax.axis_index("s") * info.num_cores + lax.axis_index("c")
        base = wid * per_w
        rows = (rows0_v, rows1_v)
        gsem = (gsem0, gsem1)
        isem = (isem0, isem1)

        pltpu.sync_copy(pos_hbm, pos2_v.at[pl.ds(0, _SEQ)])
        pltpu.sync_copy(pos_hbm, pos2_v.at[pl.ds(_SEQ, _SEQ)])

        def idx_copy(b, g):
            return pltpu.make_async_copy(
                x_hbm.at[pl.ds(base + g * _CHUNK, _CHUNK)],
                idx_v.at[b], isem[b])

        def gather(b, g):
            del g
            return pltpu.make_async_copy(
                table_hbm.at[idx_v.at[b]], rows[b], gsem[b])

        idx_copy(0, 0).start()
        idx_copy(1, 1).start()
        idx_copy(0, 0).wait()
        gather(0, 0).start()

        def step(b, g):
            gather(b, g).wait()

            @pl.when(g + 2 < n_chunks)
            def _():
                idx_copy(b, g + 2).start()

            @pl.when(g + 1 < n_chunks)
            def _():
                idx_copy(1 - b, g + 1).wait()
                gather(1 - b, g + 1).start()

            def add_row(i, c):
                for j in range(_D // _LANES):
                    plsc.addupdate(
                        rows[b].at[i, pl.ds(j * _LANES, _LANES)],
                        pos2_v[i, pl.ds(j * _LANES, _LANES)],
                    )
                return c

            lax.fori_loop(0, _CHUNK, add_row, 0, unroll=4)
            pltpu.sync_copy(rows[b], out_hbm.at[pl.ds(base + g * _CHUNK, _CHUNK)])

        def pair(t, carry):
            step(0, 2 * t)
            step(1, 2 * t + 1)
            return carry

        lax.fori_loop(0, n_chunks // 2, pair, 0)

    return emb


def kernel(x, token_table, pos_table):
    b, l = x.shape
    out = _build(b * l)(x.reshape(-1), token_table, pos_table)
    return out.reshape(b, l, _D)
